# Initial kernel scaffold; baseline (speedup 1.0000x reference)
import jax
import jax.numpy as jnp
from jax.experimental import pallas as pl


def kernel(x, edge_index, edge_weight, batch, c1_w1, c1_b1, c1_w2, c1_b2, c1_root, c1_bias, c2_w1, c2_b1, c2_w2, c2_b2, c2_root, c2_bias, c3_w1, c3_b1, c3_w2, c3_b2, c3_root, c3_bias, c4_w1, c4_b1, c4_w2, c4_b2, c4_root, c4_bias, c5_w1, c5_b1, c5_w2, c5_b2, c5_root, c5_bias, l1_w, l1_b, l2_w, l2_b, l3_w, l3_b):
    raise NotImplementedError("write your pallas kernel here")


# speedup vs baseline: 30.9841x; 30.9841x over previous
import jax
import jax.numpy as jnp
from jax import lax
from jax.experimental import pallas as pl
from jax.experimental.pallas import tpu as pltpu
from jax.experimental.pallas import tpu_sc as plsc

N_NODES = 100000
N_EDGES = 1600000
N_GRAPHS = 64

NC = 2
NS = 16
L = 16

NTILES = NC * NS
W = 5120
CHUNK = 128
EPT = 51200
NWIN = EPT // W
E_PAD = NTILES * EPT

NODE_CH = 6256
N_PAD = NS * NODE_CH
DENSE_CHUNKS = [(0, 2048), (2048, 2048), (4096, 2048), (6144, 112)]

_LAYER_DIMS = [
    (2, 2, 2),
    (2, 3, 3),
    (3, 3, 3),
    (3, 2, 3),
    (2, 1, 2),
]


def _mesh():
  return plsc.VectorSubcoreMesh(
      core_axis_name="c", subcore_axis_name="s", num_cores=NC, num_subcores=NS
  )


def _relu(v):
  return jnp.maximum(v, 0.0)


def _make_layer_kernel(layer_idx):
  ic, oc, hh = _LAYER_DIMS[layer_idx]
  first = layer_idx == 0
  if first:
    icp = ocp = None
  else:
    icp, ocp, _ = _LAYER_DIMS[layer_idx - 1]
    assert ocp == ic

  off = 0
  if not first:
    p_root = off; off += icp * ocp
    p_biasd = off; off += ocp
  p_w1 = off; off += hh
  p_b1 = off; off += hh
  p_w2 = off; off += ic * oc * hh
  p_b2 = off; off += ic * oc
  n_params = off

  def body(*refs):
    it = iter(refs)
    if first:
      xprev = [next(it) for _ in range(2)]
      aggprev = None
    else:
      xprev = [next(it) for _ in range(icp)]
      aggprev = [[next(it) for _ in range(ocp)] for _ in range(NC)]
    src_h = next(it)
    dst_h = next(it)
    ew_h = next(it)
    pb_h = next(it)
    if first:
      xout = None
    else:
      xout = [next(it) for _ in range(ic)]
    aggout = next(it)
    xtab = [next(it) for _ in range(ic)]
    aggsh = [next(it) for _ in range(oc)]
    pbuf = next(it)
    srcb = next(it)
    dstb = next(it)
    ewb = next(it)
    xgb = next(it)
    mb = next(it)
    dx = next(it)
    da = next(it)
    dob = next(it)

    c = lax.axis_index("c")
    s = lax.axis_index("s")
    wid = c * NS + s
    nbase = s * NODE_CH

    pltpu.sync_copy(pb_h, pbuf)

    def pv(i):
      return pbuf[pl.ds(i * L, L)]

    zeros = jnp.zeros((L,), jnp.float32)
    for coff, csz in DENSE_CHUNKS:
      if first:
        for i in range(2):
          pltpu.sync_copy(
              xprev[i].at[pl.ds(nbase + coff, csz)], dx.at[pl.ds(0, csz)]
          )
          pltpu.sync_copy(
              dx.at[pl.ds(0, csz)], xtab[i].at[pl.ds(nbase + coff, csz)]
          )
      else:
        for i in range(icp):
          pltpu.sync_copy(
              xprev[i].at[pl.ds(nbase + coff, csz)],
              dx.at[pl.ds(i * 2048, csz)],
          )
        for p in range(NC):
          for o in range(ocp):
            pltpu.sync_copy(
                aggprev[p][o].at[pl.ds(nbase + coff, csz)],
                da.at[pl.ds((p * ocp + o) * 2048, csz)],
            )
        roots = [[pv(p_root + i * ocp + o) for o in range(ocp)]
                 for i in range(icp)]
        biasd = [pv(p_biasd + o) for o in range(ocp)]

        def dense_g(k, _):
          q = k * L
          xs = [dx[pl.ds(i * 2048 + q, L)] for i in range(icp)]
          for o in range(ocp):
            acc = (da[pl.ds(o * 2048 + q, L)]
                   + da[pl.ds((ocp + o) * 2048 + q, L)] + biasd[o])
            for i in range(icp):
              acc = acc + xs[i] * roots[i][o]
            dob[pl.ds(o * 2048 + q, L)] = _relu(acc)
          return 0

        lax.fori_loop(0, csz // L, dense_g, 0)
        for o in range(ocp):
          pltpu.sync_copy(
              dob.at[pl.ds(o * 2048, csz)],
              xtab[o].at[pl.ds(nbase + coff, csz)],
          )
          pltpu.sync_copy(
              dob.at[pl.ds(o * 2048, csz)],
              xout[o].at[pl.ds(nbase + coff, csz)],
          )

    def zero_g(k, _):
      dx[pl.ds(k * L, L)] = zeros
      return 0

    lax.fori_loop(0, 2048 // L, zero_g, 0)
    for o in range(oc):
      for coff, csz in DENSE_CHUNKS:
        pltpu.sync_copy(
            dx.at[pl.ds(0, csz)], aggsh[o].at[pl.ds(nbase + coff, csz)]
        )

    plsc.subcore_barrier()

    w1 = [pv(p_w1 + j) for j in range(hh)]
    b1 = [pv(p_b1 + j) for j in range(hh)]
    w2 = [[pv(p_w2 + k * hh + j) for j in range(hh)] for k in range(ic * oc)]
    b2 = [pv(p_b2 + k) for k in range(ic * oc)]

    ebase = wid * EPT

    def wbody(w, _):
      pltpu.sync_copy(src_h.at[pl.ds(ebase + w * W, W)], srcb)
      pltpu.sync_copy(dst_h.at[pl.ds(ebase + w * W, W)], dstb)
      pltpu.sync_copy(ew_h.at[pl.ds(ebase + w * W, W)], ewb)
      for i in range(ic):
        pltpu.sync_copy(xtab[i].at[srcb], xgb.at[pl.ds(i * W, W)])

      def gbody(k, _):
        q = k * L
        ewv = ewb[pl.ds(q, L)]
        h1 = [_relu(ewv * w1[j] + b1[j]) for j in range(hh)]
        h2 = []
        for kk in range(ic * oc):
          acc = b2[kk]
          for j in range(hh):
            acc = acc + h1[j] * w2[kk][j]
          h2.append(_relu(acc))
        xs = [xgb[pl.ds(i * W + q, L)] for i in range(ic)]
        for o in range(oc):
          m = xs[0] * h2[o]
          for i in range(1, ic):
            m = m + xs[i] * h2[i * oc + o]
          mb[pl.ds(o * W + q, L)] = m
        return 0

      lax.fori_loop(0, W // L, gbody, 0)
      for o in range(oc):
        pltpu.sync_copy(
            mb.at[pl.ds(o * W, W)],
            aggsh[o].at[plsc.Indices(dstb, ignored_value=-1)],
            add=True,
        )
      return 0

    lax.fori_loop(0, NWIN, wbody, 0)

    plsc.subcore_barrier()

    for o in range(oc):
      row = (c * oc + o) * N_PAD
      for coff, csz in DENSE_CHUNKS:
        pltpu.sync_copy(
            aggsh[o].at[pl.ds(nbase + coff, csz)], dx.at[pl.ds(0, csz)]
        )
        pltpu.sync_copy(
            dx.at[pl.ds(0, csz)],
            aggout.at[pl.ds(row + nbase + coff, csz)],
        )

  out_type = []
  if not first:
    out_type += [jax.ShapeDtypeStruct((N_PAD,), jnp.float32)] * ic
  out_type += [jax.ShapeDtypeStruct((NC * oc * N_PAD,), jnp.float32)]

  scratch = (
      [pltpu.VMEM_SHARED((N_PAD,), jnp.float32)] * ic
      + [pltpu.VMEM_SHARED((N_PAD,), jnp.float32)] * oc
      + [
          pltpu.VMEM((n_params * L,), jnp.float32),
          pltpu.VMEM((W,), jnp.int32),
          pltpu.VMEM((W,), jnp.int32),
          pltpu.VMEM((W,), jnp.float32),
          pltpu.VMEM((ic * W,), jnp.float32),
          pltpu.VMEM((oc * W,), jnp.float32),
          pltpu.VMEM(((2 if first else icp) * 2048,), jnp.float32),
          pltpu.VMEM(((1 if first else NC * ocp) * 2048,), jnp.float32),
          pltpu.VMEM(((2 if first else ocp) * 2048,), jnp.float32),
      ]
  )

  return pl.kernel(
      body,
      out_type=tuple(out_type),
      mesh=_mesh(),
      scratch_types=scratch,
      name=f"nnconv_layer{layer_idx + 1}",
  ), n_params


def _make_tail_kernel():
  icp, ocp = 2, 1
  off = 0
  p_root = off; off += icp * ocp
  p_biasd = off; off += ocp
  p_l1w = off; off += 7 * 13
  p_l1b = off; off += 7
  p_l2w = off; off += 4 * 7
  p_l2b = off; off += 4
  p_l3w = off; off += 4
  p_l3b = off; off += 1
  n_params = off

  NF = 13

  def body(*refs):
    it = iter(refs)
    xcols = [next(it) for _ in range(12)]
    agg5 = [[next(it)] for _ in range(NC)]
    batch_h = next(it)
    pb_h = next(it)
    out_h = next(it)
    sums_sh = next(it)
    cnts_sh = next(it)
    pbuf = next(it)
    fb = next(it)
    bb = next(it)
    vb = next(it)
    ib = next(it)
    ob = next(it)
    rb = next(it)
    outb = next(it)

    c = lax.axis_index("c")
    s = lax.axis_index("s")
    nbase = s * NODE_CH

    @pl.when(c == 0)
    def _work():
      pltpu.sync_copy(pb_h, pbuf)

      def pv(i):
        return pbuf[pl.ds(i * L, L)]

      def zg(k, _):
        vb[pl.ds(k * L, L)] = jnp.zeros((L,), jnp.float32)
        return 0

      lax.fori_loop(0, 64 // L, zg, 0)
      pltpu.sync_copy(vb.at[pl.ds(0, 64)], sums_sh.at[pl.ds(s * 64, 64)])
      pltpu.sync_copy(vb.at[pl.ds(0, 64)], cnts_sh.at[pl.ds(s * 64, 64)])

      def og(k, _):
        ob[pl.ds(k * L, L)] = jnp.full((L,), 1.0, jnp.float32)
        return 0

      lax.fori_loop(0, 2048 // L, og, 0)

      root5 = [pv(p_root + i) for i in range(icp)]
      bias5 = pv(p_biasd)
      l1w = [[pv(p_l1w + j * 13 + i) for i in range(13)] for j in range(7)]
      l1b = [pv(p_l1b + j) for j in range(7)]
      l2w = [[pv(p_l2w + j * 7 + i) for i in range(7)] for j in range(4)]
      l2b = [pv(p_l2b + j) for j in range(4)]
      l3w = [pv(p_l3w + i) for i in range(4)]
      l3b = pv(p_l3b)

      for coff, csz in DENSE_CHUNKS:
        for i in range(12):
          pltpu.sync_copy(
              xcols[i].at[pl.ds(nbase + coff, csz)],
              fb.at[pl.ds(i * 2048, csz)],
          )
        pltpu.sync_copy(
            agg5[0][0].at[pl.ds(nbase + coff, csz)],
            fb.at[pl.ds(12 * 2048, csz)],
        )
        pltpu.sync_copy(
            batch_h.at[pl.ds(nbase + coff, csz)], bb.at[pl.ds(0, csz)]
        )

        def ng(k, _):
          q = k * L
          a1 = fb[pl.ds(12 * 2048 + q, L)]
          a2 = vb[pl.ds(q, L)]
          x5a = fb[pl.ds(10 * 2048 + q, L)]
          x5b = fb[pl.ds(11 * 2048 + q, L)]
          x6 = _relu(a1 + a2 + bias5 + x5a * root5[0] + x5b * root5[1])
          f = [fb[pl.ds(i * 2048 + q, L)] for i in range(12)] + [x6]
          h1 = []
          for j in range(7):
            acc = l1b[j]
            for i in range(13):
              acc = acc + f[i] * l1w[j][i]
            h1.append(_relu(acc))
          h2 = []
          for j in range(4):
            acc = l2b[j]
            for i in range(7):
              acc = acc + h1[i] * l2w[j][i]
            h2.append(_relu(acc))
          acc = l3b
          for i in range(4):
            acc = acc + h2[i] * l3w[i]
          val = _relu(acc)
          bv = bb[pl.ds(q, L)]
          sidx = jnp.where(bv < 0, -1, bv + s * 64)
          ib[pl.ds(q, L)] = sidx
          vb[pl.ds(q, L)] = val
          return 0

        pltpu.sync_copy(
            agg5[1][0].at[pl.ds(nbase + coff, csz)], vb.at[pl.ds(0, csz)]
        )
        lax.fori_loop(0, csz // L, ng, 0)
        for ch in range((csz + CHUNK - 1) // CHUNK):
          cs = min(CHUNK, csz - ch * CHUNK)
          pltpu.sync_copy(
              vb.at[pl.ds(ch * CHUNK, cs)],
              sums_sh.at[plsc.Indices(ib.at[pl.ds(ch * CHUNK, cs)],
                                      ignored_value=-1)],
              add=True,
          )
          pltpu.sync_copy(
              ob.at[pl.ds(ch * CHUNK, cs)],
              cnts_sh.at[plsc.Indices(ib.at[pl.ds(ch * CHUNK, cs)],
                                      ignored_value=-1)],
              add=True,
          )

    plsc.subcore_barrier()

    @pl.when(jnp.logical_and(c == 0, s == 0))
    def _finalize():
      pltpu.sync_copy(sums_sh, rb.at[pl.ds(0, NS * 64)])
      pltpu.sync_copy(cnts_sh, rb.at[pl.ds(NS * 64, NS * 64)])
      for q in range(64 // L):
        acc = jnp.zeros((L,), jnp.float32)
        cnt = jnp.zeros((L,), jnp.float32)
        for r in range(NS):
          acc = acc + rb[pl.ds(r * 64 + q * L, L)]
          cnt = cnt + rb[pl.ds(NS * 64 + r * 64 + q * L, L)]
        outb[pl.ds(q * L, L)] = _relu(acc / jnp.maximum(cnt, 1.0))
      pltpu.sync_copy(outb, out_h)

  scratch = [
      pltpu.VMEM_SHARED((NS * 64,), jnp.float32),
      pltpu.VMEM_SHARED((NS * 64,), jnp.float32),
      pltpu.VMEM((n_params * L,), jnp.float32),
      pltpu.VMEM((NF * 2048,), jnp.float32),
      pltpu.VMEM((2048,), jnp.int32),
      pltpu.VMEM((2048,), jnp.float32),
      pltpu.VMEM((2048,), jnp.int32),
      pltpu.VMEM((2048,), jnp.float32),
      pltpu.VMEM((2 * NS * 64,), jnp.float32),
      pltpu.VMEM((64,), jnp.float32),
  ]

  return pl.kernel(
      body,
      out_type=jax.ShapeDtypeStruct((N_GRAPHS,), jnp.float32),
      mesh=_mesh(),
      scratch_types=scratch,
      name="gnn_tail",
  ), n_params


def _pad_col(col):
  return jnp.pad(col, (0, N_PAD - N_NODES))


def kernel(x, edge_index, edge_weight, batch, c1_w1, c1_b1, c1_w2, c1_b2, c1_root, c1_bias, c2_w1, c2_b1, c2_w2, c2_b2, c2_root, c2_bias, c3_w1, c3_b1, c3_w2, c3_b2, c3_root, c3_bias, c4_w1, c4_b1, c4_w2, c4_b2, c4_root, c4_bias, c5_w1, c5_b1, c5_w2, c5_b2, c5_root, c5_bias, l1_w, l1_b, l2_w, l2_b, l3_w, l3_b):
  convs = [
      (c1_w1, c1_b1, c1_w2, c1_b2, c1_root, c1_bias),
      (c2_w1, c2_b1, c2_w2, c2_b2, c2_root, c2_bias),
      (c3_w1, c3_b1, c3_w2, c3_b2, c3_root, c3_bias),
      (c4_w1, c4_b1, c4_w2, c4_b2, c4_root, c4_bias),
      (c5_w1, c5_b1, c5_w2, c5_b2, c5_root, c5_bias),
  ]

  src = jnp.concatenate(
      [edge_index[0], jnp.zeros((E_PAD - N_EDGES,), jnp.int32)]
  )
  dst = jnp.concatenate(
      [edge_index[1], jnp.full((E_PAD - N_EDGES,), -1, jnp.int32)]
  )
  ew = jnp.concatenate(
      [edge_weight, jnp.zeros((E_PAD - N_EDGES,), jnp.float32)]
  )
  batch_p = jnp.concatenate(
      [batch, jnp.full((N_PAD - N_NODES,), -1, jnp.int32)]
  )

  xcols = [_pad_col(x[:, 0]), _pad_col(x[:, 1])]

  def pbcast(arrays):
    flat = jnp.concatenate([a.reshape(-1) for a in arrays])
    return jnp.broadcast_to(flat[:, None], (flat.shape[0], L)).reshape(-1)

  cur_cols = xcols
  all_cols = list(xcols)
  aggs = None
  for li in range(5):
    kfn, n_params = _make_layer_kernel(li)
    w1, b1, w2, b2, root, bias = convs[li]
    oc = _LAYER_DIMS[li][1]
    if li == 0:
      pb = pbcast([w1, b1, w2, b2])
      aggf = kfn(*cur_cols, src, dst, ew, pb)
      aggf = aggf[0] if isinstance(aggf, (tuple, list)) else aggf
    else:
      proot, pbias = convs[li - 1][4], convs[li - 1][5]
      pb = pbcast([proot, pbias, w1, b1, w2, b2])
      outs = kfn(*cur_cols, *aggs, src, dst, ew, pb)
      ic = _LAYER_DIMS[li][0]
      cur_cols = list(outs[:ic])
      all_cols.extend(cur_cols)
      aggf = outs[ic]
    aggs = [aggf[r * N_PAD:(r + 1) * N_PAD] for r in range(NC * oc)]

  tkfn, _ = _make_tail_kernel()
  proot, pbias = convs[4][4], convs[4][5]
  pbt = pbcast([proot, pbias, l1_w, l1_b, l2_w, l2_b, l3_w, l3_b])
  out = tkfn(*all_cols, aggs[0], aggs[1], batch_p, pbt)
  return out
